# baseline (speedup 1.0000x reference)
import jax
import jax.numpy as jnp
from jax.experimental import pallas as pl

_N = 256
_BB = 8


def _chunk_offsets(n):
    offs = [0] * n
    for d in range(1, n):
        offs[d] = (d - 1) * n - (d * (d - 1)) // 2
    return offs


_OFFS = _chunk_offsets(_N)


def _fill_kernel(v_ref, u_ref, o_ref):
    v = v_ref[...]
    rows = [jnp.zeros((_BB, 1, _N), jnp.float32)]
    for d in range(1, _N):
        off = _OFFS[d]
        rows.append(v[:, off:off + _N][:, None, :])
    p = jnp.concatenate(rows, axis=1)

    lane = jax.lax.broadcasted_iota(jnp.int32, (_N, _N), 1)
    for i in range(8):
        s = 1 << i
        rolled = jnp.roll(p, s, axis=1)
        m = ((lane >> i) & 1) == 1
        p = jnp.where(m[None], rolled, p)

    row = jax.lax.broadcasted_iota(jnp.int32, (_N, _N), 0)
    tri = row > lane
    o_ref[...] = jnp.where(tri[None], p, 0.0) + u_ref[...][None]


def kernel(inputs, unit_matrix):
    b = inputs.shape[0]
    vpad = jnp.pad(inputs, ((0, 0), (0, _N)))
    vl = vpad.shape[1]
    return pl.pallas_call(
        _fill_kernel,
        grid=(b // _BB,),
        in_specs=[
            pl.BlockSpec((_BB, vl), lambda i: (i, 0)),
            pl.BlockSpec((_N, _N), lambda i: (0, 0)),
        ],
        out_specs=pl.BlockSpec((_BB, _N, _N), lambda i: (i, 0, 0)),
        out_shape=jax.ShapeDtypeStruct((b, _N, _N), inputs.dtype),
    )(vpad, unit_matrix)
